# baseline (speedup 1.0000x reference)
import functools
import jax
import jax.numpy as jnp
from jax import lax
from jax.experimental import pallas as pl
from jax.experimental.pallas import tpu as pltpu
from jax.experimental.pallas import tpu_sc as plsc

_B, _T, _D = 4, 8192, 1024
_NW = 32
_TPW = _T // _NW
_C = 32
_NCH = _TPW // _C
_STEPS = _NCH * _B


def _sc_body(x_hbm, pos_hbm, out_hbm, xv0, xv1, pv, sx0, sx1, so0, so1, sp):
    wid = lax.axis_index("s") * 2 + lax.axis_index("c")
    t0 = wid * _TPW
    xv, sx, so = (xv0, xv1), (sx0, sx1), (so0, so1)

    def x_slice(s):
        ci, b = divmod(s, _B)
        row = b * _T + t0 + ci * _C
        return x_hbm.at[pl.ds(row * _D, _C * _D)]

    def out_slice(s):
        ci, b = divmod(s, _B)
        row = b * _T + t0 + ci * _C
        return out_hbm.at[pl.ds(row * _D, _C * _D)]

    def pos_slice(ci):
        return pos_hbm.at[pl.ds((t0 + ci * _C) * _D, _C * _D)]

    pltpu.make_async_copy(pos_slice(0), pv, sp).start()
    pltpu.make_async_copy(x_slice(0), xv[0], sx[0]).start()

    for s in range(_STEPS):
        p = s % 2
        ci, b = divmod(s, _B)
        pltpu.make_async_copy(x_slice(s), xv[p], sx[p]).wait()
        if b == 0:
            pltpu.make_async_copy(pos_slice(ci), pv, sp).wait()
        if s + 1 < _STEPS:
            q = (s + 1) % 2
            if s >= 1:
                pltpu.make_async_copy(xv[q], out_slice(s - 1), so[q]).wait()
            pltpu.make_async_copy(x_slice(s + 1), xv[q], sx[q]).start()

        xbuf = xv[p]

        @plsc.parallel_loop(0, _C * _D // 16, unroll=8)
        def add_loop(i):
            sl = pl.ds(i * 16, 16)
            xbuf[sl] = xbuf[sl] + pv[sl]

        pltpu.make_async_copy(xbuf, out_slice(s), so[p]).start()
        if b == _B - 1 and ci + 1 < _NCH:
            pltpu.make_async_copy(pos_slice(ci + 1), pv, sp).start()

    pltpu.make_async_copy(xv[(_STEPS - 2) % 2], out_slice(_STEPS - 2),
                          so[(_STEPS - 2) % 2]).wait()
    pltpu.make_async_copy(xv[(_STEPS - 1) % 2], out_slice(_STEPS - 1),
                          so[(_STEPS - 1) % 2]).wait()


def kernel(x, pos_table):
    B, T, D = x.shape
    mesh = plsc.VectorSubcoreMesh(
        core_axis_name="c", subcore_axis_name="s", num_cores=2, num_subcores=16
    )
    body = functools.partial(
        pl.kernel,
        mesh=mesh,
        out_type=jax.ShapeDtypeStruct((B * T * D,), jnp.float32),
        scratch_types=[
            pltpu.VMEM((_C * _D,), jnp.float32),
            pltpu.VMEM((_C * _D,), jnp.float32),
            pltpu.VMEM((_C * _D,), jnp.float32),
            pltpu.SemaphoreType.DMA,
            pltpu.SemaphoreType.DMA,
            pltpu.SemaphoreType.DMA,
            pltpu.SemaphoreType.DMA,
            pltpu.SemaphoreType.DMA,
        ],
    )(_sc_body)
    out = body(x.reshape(-1), pos_table.reshape(-1))
    return out.reshape(x.shape)
